# baseline (speedup 1.0000x reference)
import functools

import jax
import jax.numpy as jnp
from jax import lax
from jax.experimental import pallas as pl
from jax.experimental.pallas import tpu as pltpu

_D = 64
_K = 1024
_TN = 512


def _vq_body(x_ref, cb_ref, xsq_ref, csq_ref, idx_ref, q_ref, msum_ref):
    x = x_ref[...]
    cb = cb_ref[...]
    s = lax.dot_general(x, cb, (((1,), (1,)), ((), ())),
                        preferred_element_type=jnp.float32)
    dist = xsq_ref[...] - 2.0 * s + csq_ref[...]
    m = jnp.min(dist, axis=1, keepdims=True)
    iota = lax.broadcasted_iota(jnp.int32, dist.shape, 1)
    idx = jnp.min(jnp.where(dist == m, iota, _K), axis=1)
    idx_ref[...] = idx
    onehot = (iota == idx[:, None]).astype(jnp.float32)
    q_ref[...] = lax.dot_general(onehot, cb, (((1,), (0,)), ((), ())),
                                 preferred_element_type=jnp.float32,
                                 precision=lax.Precision.HIGHEST)

    @pl.when(pl.program_id(0) == 0)
    def _():
        msum_ref[0, 0] = 0.0

    msum_ref[0, 0] += jnp.sum(m)


def kernel(inputs, codebook):
    n = inputs.shape[0]
    flat = inputs.reshape(-1, _D)
    x_sq = jnp.sum(flat ** 2, axis=1, keepdims=True)
    c_sq = jnp.sum(codebook ** 2, axis=1)[None, :]
    grid = n // _TN
    idx, q, msum = pl.pallas_call(
        _vq_body,
        grid=(grid,),
        in_specs=[
            pl.BlockSpec((_TN, _D), lambda i: (i, 0)),
            pl.BlockSpec((_K, _D), lambda i: (0, 0)),
            pl.BlockSpec((_TN, 1), lambda i: (i, 0)),
            pl.BlockSpec((1, _K), lambda i: (0, 0)),
        ],
        out_specs=[
            pl.BlockSpec((_TN,), lambda i: (i,)),
            pl.BlockSpec((_TN, _D), lambda i: (i, 0)),
            pl.BlockSpec((1, 1), lambda i: (0, 0), memory_space=pltpu.SMEM),
        ],
        out_shape=[
            jax.ShapeDtypeStruct((n,), jnp.int32),
            jax.ShapeDtypeStruct((n, _D), jnp.float32),
            jax.ShapeDtypeStruct((1, 1), jnp.float32),
        ],
    )(flat, codebook, x_sq, c_sq)
    mse = msum[0, 0] / (n * _D)
    loss = mse + 0.25 * mse
    return loss, q, idx


# speedup vs baseline: 1.3601x; 1.3601x over previous
import functools

import jax
import jax.numpy as jnp
from jax import lax
from jax.experimental import pallas as pl
from jax.experimental.pallas import tpu as pltpu
from jax.experimental.pallas import tpu_sc as plsc

_D = 64
_K = 1024
_TN = 512


def _vq_body(x_ref, cb_ref, xsq_ref, csq_ref, idx_ref, msum_ref):
    x = x_ref[...]
    cb = cb_ref[...]
    s = lax.dot_general(x, cb, (((1,), (1,)), ((), ())),
                        preferred_element_type=jnp.float32)
    dist = xsq_ref[...] - 2.0 * s + csq_ref[...]
    m = jnp.min(dist, axis=1, keepdims=True)
    iota = lax.broadcasted_iota(jnp.int32, dist.shape, 1)
    idx_ref[...] = jnp.min(jnp.where(dist == m, iota, _K), axis=1)

    @pl.when(pl.program_id(0) == 0)
    def _():
        msum_ref[0, 0] = 0.0

    msum_ref[0, 0] += jnp.sum(m)


def _argmin_and_loss(flat, codebook):
    n = flat.shape[0]
    x_sq = jnp.sum(flat ** 2, axis=1, keepdims=True)
    c_sq = jnp.sum(codebook ** 2, axis=1)[None, :]
    idx, msum = pl.pallas_call(
        _vq_body,
        grid=(n // _TN,),
        in_specs=[
            pl.BlockSpec((_TN, _D), lambda i: (i, 0)),
            pl.BlockSpec((_K, _D), lambda i: (0, 0)),
            pl.BlockSpec((_TN, 1), lambda i: (i, 0)),
            pl.BlockSpec((1, _K), lambda i: (0, 0)),
        ],
        out_specs=[
            pl.BlockSpec((_TN,), lambda i: (i,)),
            pl.BlockSpec((1, 1), lambda i: (0, 0), memory_space=pltpu.SMEM),
        ],
        out_shape=[
            jax.ShapeDtypeStruct((n,), jnp.int32),
            jax.ShapeDtypeStruct((1, 1), jnp.float32),
        ],
    )(flat, codebook, x_sq, c_sq)
    return idx, msum


def _make_sc_gather(n):
    info = plsc.get_sparse_core_info()
    nw = info.num_cores * info.num_subcores
    b_per_w = n // nw
    chunk = 96
    nch = b_per_w // chunk
    mesh = plsc.VectorSubcoreMesh(core_axis_name="c", subcore_axis_name="s")

    @functools.partial(
        pl.kernel, mesh=mesh,
        compiler_params=pltpu.CompilerParams(use_tc_tiling_on_sc=False),
        out_type=jax.ShapeDtypeStruct((n, _D), jnp.float32),
        scratch_types=[
            pltpu.VMEM((b_per_w,), jnp.int32),
            pltpu.VMEM((b_per_w, _D), jnp.float32),
            pltpu.SemaphoreType.DMA,
        ],
    )
    def gather_k(table_hbm, idx_hbm, out_hbm, idx_v, rows_v, sem):
        wid = lax.axis_index("s") * info.num_cores + lax.axis_index("c")
        base = wid * b_per_w
        pltpu.sync_copy(idx_hbm.at[pl.ds(base, b_per_w)], idx_v)
        copies = []
        for j in range(nch):
            copies.append(pltpu.async_copy(
                table_hbm.at[idx_v.at[pl.ds(j * chunk, chunk)]],
                rows_v.at[pl.ds(j * chunk, chunk)], sem))
        for c in copies:
            c.wait()
        pltpu.sync_copy(rows_v, out_hbm.at[pl.ds(base, b_per_w)])

    return gather_k


def kernel(inputs, codebook):
    n = inputs.shape[0]
    flat = inputs.reshape(-1, _D)
    idx, msum = _argmin_and_loss(flat, codebook)
    quantized = _make_sc_gather(n)(codebook, idx)
    mse = msum[0, 0] / (n * _D)
    loss = mse + 0.25 * mse
    return loss, quantized, idx


# speedup vs baseline: 1.5809x; 1.1623x over previous
import functools

import jax
import jax.numpy as jnp
from jax import lax
from jax.experimental import pallas as pl
from jax.experimental.pallas import tpu as pltpu
from jax.experimental.pallas import tpu_sc as plsc

_D = 64
_K = 1024
_TN = 512


def _vq_body(x_ref, cb2_ref, xsq_ref, csq_ref, idx_ref, msum_ref):
    x = x_ref[...]
    cb2 = cb2_ref[...]
    s2 = lax.dot_general(cb2, x, (((1,), (1,)), ((), ())),
                         preferred_element_type=jnp.float32)
    dist = (xsq_ref[...] + s2) + csq_ref[...]
    m = jnp.min(dist, axis=0, keepdims=True)
    iota = lax.broadcasted_iota(jnp.int32, dist.shape, 0)
    idx_ref[...] = jnp.min(jnp.where(dist == m, iota, _K), axis=0)

    @pl.when(pl.program_id(0) == 0)
    def _():
        msum_ref[0, 0] = 0.0

    msum_ref[0, 0] += jnp.sum(m)


def _argmin_and_loss(flat, codebook):
    n = flat.shape[0]
    x_sq = jnp.sum(flat ** 2, axis=1, keepdims=True)
    c_sq = jnp.sum(codebook ** 2, axis=1)[:, None]
    idx, msum = pl.pallas_call(
        _vq_body,
        grid=(n // _TN,),
        in_specs=[
            pl.BlockSpec((_TN, _D), lambda i: (i, 0)),
            pl.BlockSpec((_K, _D), lambda i: (0, 0)),
            pl.BlockSpec((1, _TN), lambda i: (0, i)),
            pl.BlockSpec((_K, 1), lambda i: (0, 0)),
        ],
        out_specs=[
            pl.BlockSpec((_TN,), lambda i: (i,)),
            pl.BlockSpec((1, 1), lambda i: (0, 0), memory_space=pltpu.SMEM),
        ],
        out_shape=[
            jax.ShapeDtypeStruct((n,), jnp.int32),
            jax.ShapeDtypeStruct((1, 1), jnp.float32),
        ],
    )(flat, -2.0 * codebook, x_sq.reshape(1, n), c_sq)
    return idx, msum


def _make_sc_gather(n):
    info = plsc.get_sparse_core_info()
    nw = info.num_cores * info.num_subcores
    b_per_w = n // nw
    chunk = 96
    nch = b_per_w // chunk
    mesh = plsc.VectorSubcoreMesh(core_axis_name="c", subcore_axis_name="s")

    @functools.partial(
        pl.kernel, mesh=mesh,
        compiler_params=pltpu.CompilerParams(use_tc_tiling_on_sc=False),
        out_type=jax.ShapeDtypeStruct((n, _D), jnp.float32),
        scratch_types=[
            pltpu.VMEM((b_per_w,), jnp.int32),
            pltpu.VMEM((b_per_w, _D), jnp.float32),
            pltpu.SemaphoreType.DMA,
        ],
    )
    def gather_k(table_hbm, idx_hbm, out_hbm, idx_v, rows_v, sem):
        wid = lax.axis_index("s") * info.num_cores + lax.axis_index("c")
        base = wid * b_per_w
        pltpu.sync_copy(idx_hbm.at[pl.ds(base, b_per_w)], idx_v)
        copies = []
        for j in range(nch):
            copies.append(pltpu.async_copy(
                table_hbm.at[idx_v.at[pl.ds(j * chunk, chunk)]],
                rows_v.at[pl.ds(j * chunk, chunk)], sem))
        for c in copies:
            c.wait()
        pltpu.sync_copy(rows_v, out_hbm.at[pl.ds(base, b_per_w)])

    return gather_k


def kernel(inputs, codebook):
    n = inputs.shape[0]
    flat = inputs.reshape(-1, _D)
    idx, msum = _argmin_and_loss(flat, codebook)
    quantized = _make_sc_gather(n)(codebook, idx)
    mse = msum[0, 0] / (n * _D)
    loss = mse + 0.25 * mse
    return loss, quantized, idx


# speedup vs baseline: 1.6041x; 1.0147x over previous
import functools

import jax
import jax.numpy as jnp
from jax import lax
from jax.experimental import pallas as pl
from jax.experimental.pallas import tpu as pltpu
from jax.experimental.pallas import tpu_sc as plsc

_D = 64
_K = 1024
_TN = 512


def _vq_body(n, x_ref, cb_ref, xsq_ref, csq_ref, idx_ref, loss_ref):
    x = x_ref[...]
    cb2 = -2.0 * cb_ref[...]
    s2 = lax.dot_general(cb2, x, (((1,), (1,)), ((), ())),
                         preferred_element_type=jnp.float32)
    dist = (xsq_ref[...] + s2) + csq_ref[...]
    m = jnp.min(dist, axis=0, keepdims=True)
    iota = lax.broadcasted_iota(jnp.int32, dist.shape, 0)
    idx_ref[...] = jnp.min(jnp.where(dist == m, iota, _K), axis=0)

    @pl.when(pl.program_id(0) == 0)
    def _():
        loss_ref[0, 0] = 0.0

    loss_ref[0, 0] += jnp.sum(m)

    @pl.when(pl.program_id(0) == pl.num_programs(0) - 1)
    def _():
        mse = loss_ref[0, 0] / (n * _D)
        loss_ref[0, 0] = mse + 0.25 * mse


def _argmin_and_loss(flat, codebook):
    n = flat.shape[0]
    x_sq = jnp.sum(flat ** 2, axis=1)[None, :]
    c_sq = jnp.sum(codebook ** 2, axis=1)[:, None]
    idx, loss = pl.pallas_call(
        functools.partial(_vq_body, n),
        grid=(n // _TN,),
        in_specs=[
            pl.BlockSpec((_TN, _D), lambda i: (i, 0)),
            pl.BlockSpec((_K, _D), lambda i: (0, 0)),
            pl.BlockSpec((1, _TN), lambda i: (0, i)),
            pl.BlockSpec((_K, 1), lambda i: (0, 0)),
        ],
        out_specs=[
            pl.BlockSpec((_TN,), lambda i: (i,)),
            pl.BlockSpec((1, 1), lambda i: (0, 0), memory_space=pltpu.SMEM),
        ],
        out_shape=[
            jax.ShapeDtypeStruct((n,), jnp.int32),
            jax.ShapeDtypeStruct((1, 1), jnp.float32),
        ],
    )(flat, codebook, x_sq, c_sq)
    return idx, loss


def _make_sc_gather(n):
    info = plsc.get_sparse_core_info()
    nw = info.num_cores * info.num_subcores
    b_per_w = n // nw
    chunk = 96
    nch = b_per_w // chunk
    mesh = plsc.VectorSubcoreMesh(core_axis_name="c", subcore_axis_name="s")

    @functools.partial(
        pl.kernel, mesh=mesh,
        compiler_params=pltpu.CompilerParams(use_tc_tiling_on_sc=False),
        out_type=jax.ShapeDtypeStruct((n, _D), jnp.float32),
        scratch_types=[
            pltpu.VMEM((b_per_w,), jnp.int32),
            pltpu.VMEM((b_per_w, _D), jnp.float32),
            pltpu.SemaphoreType.DMA,
        ],
    )
    def gather_k(table_hbm, idx_hbm, out_hbm, idx_v, rows_v, sem):
        wid = lax.axis_index("s") * info.num_cores + lax.axis_index("c")
        base = wid * b_per_w
        pltpu.sync_copy(idx_hbm.at[pl.ds(base, b_per_w)], idx_v)
        copies = []
        for j in range(nch):
            copies.append(pltpu.async_copy(
                table_hbm.at[idx_v.at[pl.ds(j * chunk, chunk)]],
                rows_v.at[pl.ds(j * chunk, chunk)], sem))
        for c in copies:
            c.wait()
        pltpu.sync_copy(rows_v, out_hbm.at[pl.ds(base, b_per_w)])

    return gather_k


def kernel(inputs, codebook):
    n = inputs.shape[0]
    flat = inputs.reshape(-1, _D)
    idx, loss = _argmin_and_loss(flat, codebook)
    quantized = _make_sc_gather(n)(codebook, idx)
    return loss[0, 0], quantized, idx


# speedup vs baseline: 1.6270x; 1.0143x over previous
import functools

import jax
import jax.numpy as jnp
from jax import lax
from jax.experimental import pallas as pl
from jax.experimental.pallas import tpu as pltpu
from jax.experimental.pallas import tpu_sc as plsc

_D = 64
_K = 1024
_TN = 512


def _vq_body(n, x_ref, cb_ref, idx_ref, loss_ref, csq_ref):
    x = x_ref[...]
    cb = cb_ref[...]
    cb2 = -2.0 * cb

    @pl.when(pl.program_id(0) == 0)
    def _():
        csq_ref[...] = jnp.sum(cb * cb, axis=1, keepdims=True)

    ones = jnp.ones((8, _D), jnp.float32)
    xsq = lax.dot_general(ones, x * x, (((1,), (1,)), ((), ())),
                          preferred_element_type=jnp.float32)[:1]
    s2 = lax.dot_general(cb2, x, (((1,), (1,)), ((), ())),
                         preferred_element_type=jnp.float32)
    dist = (xsq + s2) + csq_ref[...]
    m = jnp.min(dist, axis=0, keepdims=True)
    iota = lax.broadcasted_iota(jnp.int32, dist.shape, 0)
    idx_ref[...] = jnp.min(jnp.where(dist == m, iota, _K), axis=0)

    @pl.when(pl.program_id(0) == 0)
    def _():
        loss_ref[0, 0] = 0.0

    loss_ref[0, 0] += jnp.sum(m)

    @pl.when(pl.program_id(0) == pl.num_programs(0) - 1)
    def _():
        mse = loss_ref[0, 0] / (n * _D)
        loss_ref[0, 0] = mse + 0.25 * mse


def _argmin_and_loss(flat, codebook):
    n = flat.shape[0]
    idx, loss = pl.pallas_call(
        functools.partial(_vq_body, n),
        grid=(n // _TN,),
        in_specs=[
            pl.BlockSpec((_TN, _D), lambda i: (i, 0)),
            pl.BlockSpec((_K, _D), lambda i: (0, 0)),
        ],
        out_specs=[
            pl.BlockSpec((_TN,), lambda i: (i,)),
            pl.BlockSpec((1, 1), lambda i: (0, 0), memory_space=pltpu.SMEM),
        ],
        out_shape=[
            jax.ShapeDtypeStruct((n,), jnp.int32),
            jax.ShapeDtypeStruct((1, 1), jnp.float32),
        ],
        scratch_shapes=[pltpu.VMEM((_K, 1), jnp.float32)],
    )(flat, codebook)
    return idx, loss


def _make_sc_gather(n):
    info = plsc.get_sparse_core_info()
    nw = info.num_cores * info.num_subcores
    b_per_w = n // nw
    chunk = 96
    nch = b_per_w // chunk
    mesh = plsc.VectorSubcoreMesh(core_axis_name="c", subcore_axis_name="s")

    @functools.partial(
        pl.kernel, mesh=mesh,
        compiler_params=pltpu.CompilerParams(use_tc_tiling_on_sc=False),
        out_type=jax.ShapeDtypeStruct((n, _D), jnp.float32),
        scratch_types=[
            pltpu.VMEM((b_per_w,), jnp.int32),
            pltpu.VMEM((b_per_w, _D), jnp.float32),
            pltpu.SemaphoreType.DMA,
        ],
    )
    def gather_k(table_hbm, idx_hbm, out_hbm, idx_v, rows_v, sem):
        wid = lax.axis_index("s") * info.num_cores + lax.axis_index("c")
        base = wid * b_per_w
        pltpu.sync_copy(idx_hbm.at[pl.ds(base, b_per_w)], idx_v)
        copies = []
        for j in range(nch):
            copies.append(pltpu.async_copy(
                table_hbm.at[idx_v.at[pl.ds(j * chunk, chunk)]],
                rows_v.at[pl.ds(j * chunk, chunk)], sem))
        for c in copies:
            c.wait()
        pltpu.sync_copy(rows_v, out_hbm.at[pl.ds(base, b_per_w)])

    return gather_k


def kernel(inputs, codebook):
    n = inputs.shape[0]
    flat = inputs.reshape(-1, _D)
    idx, loss = _argmin_and_loss(flat, codebook)
    quantized = _make_sc_gather(n)(codebook, idx)
    return loss[0, 0], quantized, idx


# speedup vs baseline: 1.7693x; 1.0874x over previous
import functools

import jax
import jax.numpy as jnp
from jax import lax
from jax.experimental import pallas as pl
from jax.experimental.pallas import tpu as pltpu
from jax.experimental.pallas import tpu_sc as plsc

_D = 64
_K = 1024
_TN = 512


def _vq_body(n, xt_ref, cbt_ref, csq_ref, idx_ref, loss_ref):
    xt = xt_ref[...]
    cbt2 = -2.0 * cbt_ref[...]
    xsq = jnp.sum(xt * xt, axis=0, keepdims=True)
    s2 = lax.dot_general(cbt2, xt, (((0,), (0,)), ((), ())),
                         preferred_element_type=jnp.float32)
    dist = (xsq + s2) + csq_ref[...]
    m = jnp.min(dist, axis=0, keepdims=True)
    iota = lax.broadcasted_iota(jnp.int32, dist.shape, 0)
    idx_ref[...] = jnp.min(jnp.where(dist == m, iota, _K), axis=0)

    @pl.when(pl.program_id(0) == 0)
    def _():
        loss_ref[0, 0] = 0.0

    loss_ref[0, 0] += jnp.sum(m)

    @pl.when(pl.program_id(0) == pl.num_programs(0) - 1)
    def _():
        mse = loss_ref[0, 0] / (n * _D)
        loss_ref[0, 0] = mse + 0.25 * mse


def _argmin_and_loss(flat, codebook):
    n = flat.shape[0]
    c_sq = jnp.sum(codebook ** 2, axis=1)[:, None]
    idx, loss = pl.pallas_call(
        functools.partial(_vq_body, n),
        grid=(n // _TN,),
        in_specs=[
            pl.BlockSpec((_D, _TN), lambda i: (0, i)),
            pl.BlockSpec((_D, _K), lambda i: (0, 0)),
            pl.BlockSpec((_K, 1), lambda i: (0, 0)),
        ],
        out_specs=[
            pl.BlockSpec((_TN,), lambda i: (i,)),
            pl.BlockSpec((1, 1), lambda i: (0, 0), memory_space=pltpu.SMEM),
        ],
        out_shape=[
            jax.ShapeDtypeStruct((n,), jnp.int32),
            jax.ShapeDtypeStruct((1, 1), jnp.float32),
        ],
    )(flat.T, codebook.T, c_sq)
    return idx, loss


def _make_sc_gather(n):
    info = plsc.get_sparse_core_info()
    nw = info.num_cores * info.num_subcores
    b_per_w = n // nw
    chunk = 96
    nch = b_per_w // chunk
    mesh = plsc.VectorSubcoreMesh(core_axis_name="c", subcore_axis_name="s")

    @functools.partial(
        pl.kernel, mesh=mesh,
        compiler_params=pltpu.CompilerParams(use_tc_tiling_on_sc=False),
        out_type=jax.ShapeDtypeStruct((n, _D), jnp.float32),
        scratch_types=[
            pltpu.VMEM((b_per_w,), jnp.int32),
            pltpu.VMEM((b_per_w, _D), jnp.float32),
            pltpu.SemaphoreType.DMA,
        ],
    )
    def gather_k(table_hbm, idx_hbm, out_hbm, idx_v, rows_v, sem):
        wid = lax.axis_index("s") * info.num_cores + lax.axis_index("c")
        base = wid * b_per_w
        pltpu.sync_copy(idx_hbm.at[pl.ds(base, b_per_w)], idx_v)
        copies = []
        for j in range(nch):
            copies.append(pltpu.async_copy(
                table_hbm.at[idx_v.at[pl.ds(j * chunk, chunk)]],
                rows_v.at[pl.ds(j * chunk, chunk)], sem))
        for c in copies:
            c.wait()
        pltpu.sync_copy(rows_v, out_hbm.at[pl.ds(base, b_per_w)])

    return gather_k


def kernel(inputs, codebook):
    n = inputs.shape[0]
    flat = inputs.reshape(-1, _D)
    idx, loss = _argmin_and_loss(flat, codebook)
    quantized = _make_sc_gather(n)(codebook, idx)
    return loss[0, 0], quantized, idx


# speedup vs baseline: 2.0730x; 1.1717x over previous
import functools

import jax
import jax.numpy as jnp
from jax import lax
from jax.experimental import pallas as pl
from jax.experimental.pallas import tpu as pltpu
from jax.experimental.pallas import tpu_sc as plsc

_D = 64
_K = 1024
_TN = 3072


def _vq_body(n, xt_ref, cbt_ref, csq_ref, idx_ref, loss_ref):
    xt = xt_ref[...]
    cbt2 = -2.0 * cbt_ref[...]
    xsq = jnp.sum(xt * xt, axis=0, keepdims=True)
    s2 = lax.dot_general(cbt2, xt, (((0,), (0,)), ((), ())),
                         preferred_element_type=jnp.float32)
    dist = (xsq + s2) + csq_ref[...]
    m = jnp.min(dist, axis=0, keepdims=True)
    iota = lax.broadcasted_iota(jnp.int32, dist.shape, 0)
    idx_ref[...] = jnp.min(jnp.where(dist == m, iota, _K), axis=0)

    @pl.when(pl.program_id(0) == 0)
    def _():
        loss_ref[0, 0] = 0.0

    loss_ref[0, 0] += jnp.sum(m)

    @pl.when(pl.program_id(0) == pl.num_programs(0) - 1)
    def _():
        mse = loss_ref[0, 0] / (n * _D)
        loss_ref[0, 0] = mse + 0.25 * mse


def _argmin_and_loss(flat, codebook):
    n = flat.shape[0]
    c_sq = jnp.sum(codebook ** 2, axis=1)[:, None]
    idx, loss = pl.pallas_call(
        functools.partial(_vq_body, n),
        grid=(n // _TN,),
        in_specs=[
            pl.BlockSpec((_D, _TN), lambda i: (0, i)),
            pl.BlockSpec((_D, _K), lambda i: (0, 0)),
            pl.BlockSpec((_K, 1), lambda i: (0, 0)),
        ],
        out_specs=[
            pl.BlockSpec((_TN,), lambda i: (i,)),
            pl.BlockSpec((1, 1), lambda i: (0, 0), memory_space=pltpu.SMEM),
        ],
        out_shape=[
            jax.ShapeDtypeStruct((n,), jnp.int32),
            jax.ShapeDtypeStruct((1, 1), jnp.float32),
        ],
    )(flat.T, codebook.T, c_sq)
    return idx, loss


def _make_sc_gather(n):
    info = plsc.get_sparse_core_info()
    nw = info.num_cores * info.num_subcores
    b_per_w = n // nw
    chunk = 96
    nch = b_per_w // chunk
    mesh = plsc.VectorSubcoreMesh(core_axis_name="c", subcore_axis_name="s")

    @functools.partial(
        pl.kernel, mesh=mesh,
        compiler_params=pltpu.CompilerParams(use_tc_tiling_on_sc=False),
        out_type=jax.ShapeDtypeStruct((n, _D), jnp.float32),
        scratch_types=[
            pltpu.VMEM((b_per_w,), jnp.int32),
            pltpu.VMEM((b_per_w, _D), jnp.float32),
            pltpu.SemaphoreType.DMA,
        ],
    )
    def gather_k(table_hbm, idx_hbm, out_hbm, idx_v, rows_v, sem):
        wid = lax.axis_index("s") * info.num_cores + lax.axis_index("c")
        base = wid * b_per_w
        pltpu.sync_copy(idx_hbm.at[pl.ds(base, b_per_w)], idx_v)
        copies = []
        for j in range(nch):
            copies.append(pltpu.async_copy(
                table_hbm.at[idx_v.at[pl.ds(j * chunk, chunk)]],
                rows_v.at[pl.ds(j * chunk, chunk)], sem))
        for c in copies:
            c.wait()
        pltpu.sync_copy(rows_v, out_hbm.at[pl.ds(base, b_per_w)])

    return gather_k


def kernel(inputs, codebook):
    n = inputs.shape[0]
    flat = inputs.reshape(-1, _D)
    idx, loss = _argmin_and_loss(flat, codebook)
    quantized = _make_sc_gather(n)(codebook, idx)
    return loss[0, 0], quantized, idx
